# baseline (speedup 1.0000x reference)
import functools

import jax
import jax.numpy as jnp
from jax import lax
from jax.experimental import pallas as pl
from jax.experimental.pallas import tpu as pltpu
from jax.experimental.pallas import tpu_sc as plsc

_RES = 8
_NBINS = _RES ** 3
_B = 64
_N = 65536
_CLASSES = 40

_C = 8192
_NCHUNK = _N // _C
_CF = 3 * _C
_L = 16


def _sc_histogram(x2):
    info = plsc.get_sparse_core_info()
    nc, ns = info.num_cores, info.num_subcores
    nw = nc * ns
    bpw = _B // nw
    mesh = plsc.VectorSubcoreMesh(core_axis_name="c", subcore_axis_name="s")

    @functools.partial(
        pl.kernel,
        out_type=jax.ShapeDtypeStruct((_B, _NBINS), jnp.float32),
        mesh=mesh,
        compiler_params=pltpu.CompilerParams(needs_layout_passes=False),
        scratch_types=[
            pltpu.VMEM((_CF,), jnp.float32),
            pltpu.VMEM((_CF,), jnp.float32),
            pltpu.VMEM((_CF,), jnp.float32),
            pltpu.VMEM((_CF,), jnp.float32),
            pltpu.VMEM((_NBINS * _L,), jnp.float32),
            pltpu.VMEM((_NBINS,), jnp.float32),
            pltpu.SemaphoreType.DMA,
            pltpu.SemaphoreType.DMA,
            pltpu.SemaphoreType.DMA,
            pltpu.SemaphoreType.DMA,
        ],
    )
    def hist_kernel(x_hbm, out_hbm, buf0, buf1, buf2, buf3, hist, featbuf,
                    sem0, sem1, sem2, sem3):
        wid = lax.axis_index("s") * nc + lax.axis_index("c")
        bufs = (buf0, buf1, buf2, buf3)
        sems = (sem0, sem1, sem2, sem3)

        ar = jnp.arange(_L, dtype=jnp.int32)
        g0 = ar * 3
        ones = jnp.full((_L,), 1.0, jnp.float32)
        zeros = jnp.zeros((_L,), jnp.float32)
        pinf = jnp.full((_L,), jnp.inf, jnp.float32)
        ninf = jnp.full((_L,), -jnp.inf, jnp.float32)

        n_tasks = bpw * 2 * _NCHUNK

        def task_src(t):
            bi = wid * bpw + t // (2 * _NCHUNK)
            ch = t % _NCHUNK
            return x_hbm.at[bi, pl.ds(ch * _CF, _CF)]

        nbuf = 4
        depth = 3
        handles = [None] * n_tasks
        for p in range(depth):
            handles[p] = pltpu.async_copy(task_src(p), bufs[p], sems[p])

        mm = None
        params = None

        def minmax_body(buf):
            def body(j, carry):
                mn0, mn1, mn2, mx0, mx1, mx2 = carry
                base = j * 48
                v0 = buf[pl.ds(base, _L)]
                v1 = buf[pl.ds(base + 16, _L)]
                v2 = buf[pl.ds(base + 32, _L)]
                return (jnp.minimum(mn0, v0), jnp.minimum(mn1, v1),
                        jnp.minimum(mn2, v2), jnp.maximum(mx0, v0),
                        jnp.maximum(mx1, v1), jnp.maximum(mx2, v2))
            return body

        def bin_body(buf, prm):
            (sc0, off0), (sc1, off1), (sc2, off2) = prm
            def body(j, carry):
                ib = j * 48 + g0
                gx = plsc.load_gather(buf, [ib])
                gy = plsc.load_gather(buf, [ib + 1])
                gz = plsc.load_gather(buf, [ib + 2])
                ix = jnp.minimum((gx * sc0 + off0).astype(jnp.int32), _RES - 1)
                iy = jnp.minimum((gy * sc1 + off1).astype(jnp.int32), _RES - 1)
                iz = jnp.minimum((gz * sc2 + off2).astype(jnp.int32), _RES - 1)
                addr = (((((ix << 3) + iy) << 3) + iz) << 4) + ar
                plsc.addupdate_scatter(hist, [addr], ones)
                return carry
            return body

        for t in range(n_tasks):
            tb = t % (2 * _NCHUNK)
            if tb == 0:
                @plsc.parallel_loop(0, (_NBINS * _L) // _L, unroll=8)
                def _(j):
                    hist[pl.ds(j * _L, _L)] = zeros
                mm = (pinf, pinf, pinf, ninf, ninf, ninf)

            handles[t].wait()
            if t + depth < n_tasks:
                handles[t + depth] = pltpu.async_copy(
                    task_src(t + depth), bufs[(t + depth) % nbuf],
                    sems[(t + depth) % nbuf])

            buf = bufs[t % nbuf]
            if tb < _NCHUNK:
                mm = plsc.parallel_loop(0, _CF // 48, unroll=8,
                                        carry=mm)(minmax_body(buf))
                if tb == _NCHUNK - 1:
                    dnums = lax.GatherDimensionNumbers(
                        offset_dims=(), collapsed_slice_dims=(0,),
                        start_index_map=(0,))

                    def allred(v, op):
                        for s in (8, 4, 2, 1):
                            perm = (ar ^ s).reshape(_L, 1)
                            shuf = lax.gather(
                                v, perm, dnums, (1,),
                                mode=lax.GatherScatterMode.PROMISE_IN_BOUNDS)
                            v = op(v, shuf)
                        return v

                    prm = []
                    for d in range(3):
                        mn_c = [jnp.where(((ar + 16 * j) % 3) == d, mm[j],
                                          pinf) for j in range(3)]
                        mx_c = [jnp.where(((ar + 16 * j) % 3) == d,
                                          mm[3 + j], ninf) for j in range(3)]
                        mnv = allred(jnp.minimum(jnp.minimum(mn_c[0],
                                                             mn_c[1]),
                                                 mn_c[2]), jnp.minimum)
                        mxv = allred(jnp.maximum(jnp.maximum(mx_c[0],
                                                             mx_c[1]),
                                                 mx_c[2]), jnp.maximum)
                        scv = _RES / (mxv - mnv)
                        prm.append((scv, -mnv * scv))
                    params = tuple(prm)
            else:
                plsc.parallel_loop(0, _CF // 48, unroll=4,
                                   carry=jnp.int32(0))(bin_body(buf, params))
                if tb == 2 * _NCHUNK - 1:
                    def red_body(k, carry):
                        ib = k * (_L * _L) + ar * _L
                        acc = plsc.load_gather(hist, [ib])
                        for l in range(1, _L):
                            acc = acc + plsc.load_gather(hist, [ib + l])
                        featbuf[pl.ds(k * _L, _L)] = acc
                        return carry
                    plsc.parallel_loop(0, _NBINS // _L, unroll=2,
                                       carry=jnp.int32(0))(red_body)
                    bi = wid * bpw + t // (2 * _NCHUNK)
                    pltpu.sync_copy(featbuf, out_hbm.at[bi])

    return hist_kernel(x2)


def _tc_classify(feats, w, b2):
    def mm(f_ref, w_ref, b_ref, o_ref):
        acc = lax.dot_general(f_ref[...], w_ref[...],
                              (((1,), (1,)), ((), ())),
                              preferred_element_type=jnp.float32)
        o_ref[...] = acc * (1.0 / _N) + b_ref[...]

    return pl.pallas_call(
        mm,
        out_shape=jax.ShapeDtypeStruct((_B, _CLASSES), jnp.float32),
    )(feats, w, b2)


def kernel(x, W, b):
    x2 = x.reshape(_B, 3 * _N)
    feats = _sc_histogram(x2)
    return _tc_classify(feats, W, b.reshape(1, _CLASSES))


# speedup vs baseline: 3.8865x; 3.8865x over previous
import functools

import jax
import jax.numpy as jnp
from jax import lax
from jax.experimental import pallas as pl
from jax.experimental.pallas import tpu as pltpu
from jax.experimental.pallas import tpu_sc as plsc

_RES = 8
_NBINS = _RES ** 3
_B = 64
_N = 65536
_CLASSES = 40

_C = 8192
_NCHUNK = _N // _C
_L = 16
_NBUF = 4
_DEPTH = 3


def _sc_histogram(xt):
    info = plsc.get_sparse_core_info()
    nc, ns = info.num_cores, info.num_subcores
    nw = nc * ns
    bpw = _B // nw
    mesh = plsc.VectorSubcoreMesh(core_axis_name="c", subcore_axis_name="s")

    @functools.partial(
        pl.kernel,
        out_type=jax.ShapeDtypeStruct((_B, _NBINS), jnp.float32),
        mesh=mesh,
        compiler_params=pltpu.CompilerParams(needs_layout_passes=False),
        scratch_types=(
            [pltpu.VMEM((_C,), jnp.float32) for _ in range(3 * _NBUF)]
            + [pltpu.VMEM((_NBINS * _L,), jnp.float32),
               pltpu.VMEM((_NBINS,), jnp.float32)]
            + [pltpu.SemaphoreType.DMA for _ in range(3 * _NBUF)]
        ),
    )
    def hist_kernel(x_hbm, out_hbm, *scratch):
        bufs = scratch[:3 * _NBUF]
        hist = scratch[3 * _NBUF]
        featbuf = scratch[3 * _NBUF + 1]
        sems = scratch[3 * _NBUF + 2:]

        wid = lax.axis_index("s") * nc + lax.axis_index("c")

        ar = jnp.arange(_L, dtype=jnp.int32)
        ones = jnp.full((_L,), 1.0, jnp.float32)
        zeros = jnp.zeros((_L,), jnp.float32)
        pinf = jnp.full((_L,), jnp.inf, jnp.float32)
        ninf = jnp.full((_L,), -jnp.inf, jnp.float32)

        n_tasks = bpw * 2 * _NCHUNK

        def start_task(t):
            bi = wid * bpw + t // (2 * _NCHUNK)
            ch = t % _NCHUNK
            k = t % _NBUF
            return [
                pltpu.async_copy(
                    x_hbm.at[d, bi, pl.ds(ch * _C, _C)],
                    bufs[3 * k + d], sems[3 * k + d])
                for d in range(3)
            ]

        handles = [None] * n_tasks
        for p in range(_DEPTH):
            handles[p] = start_task(p)

        mm = None
        params = None

        def minmax_body(bx, by, bz):
            def body(j, carry):
                mn0, mn1, mn2, mx0, mx1, mx2 = carry
                base = j * _L
                v0 = bx[pl.ds(base, _L)]
                v1 = by[pl.ds(base, _L)]
                v2 = bz[pl.ds(base, _L)]
                return (jnp.minimum(mn0, v0), jnp.minimum(mn1, v1),
                        jnp.minimum(mn2, v2), jnp.maximum(mx0, v0),
                        jnp.maximum(mx1, v1), jnp.maximum(mx2, v2))
            return body

        def bin_body(bx, by, bz, prm):
            (sc0, off0), (sc1, off1), (sc2, off2) = prm
            def body(j, carry):
                base = j * _L
                gx = bx[pl.ds(base, _L)]
                gy = by[pl.ds(base, _L)]
                gz = bz[pl.ds(base, _L)]
                ix = jnp.minimum((gx * sc0 + off0).astype(jnp.int32), _RES - 1)
                iy = jnp.minimum((gy * sc1 + off1).astype(jnp.int32), _RES - 1)
                iz = jnp.minimum((gz * sc2 + off2).astype(jnp.int32), _RES - 1)
                addr = (((((ix << 3) + iy) << 3) + iz) << 4) + ar
                plsc.addupdate_scatter(hist, [addr], ones)
                return carry
            return body

        for t in range(n_tasks):
            tb = t % (2 * _NCHUNK)
            if tb == 0:
                @plsc.parallel_loop(0, (_NBINS * _L) // _L, unroll=8)
                def _(j):
                    hist[pl.ds(j * _L, _L)] = zeros
                mm = (pinf, pinf, pinf, ninf, ninf, ninf)

            for h in handles[t]:
                h.wait()
            if t + _DEPTH < n_tasks:
                handles[t + _DEPTH] = start_task(t + _DEPTH)

            k = t % _NBUF
            bx, by, bz = bufs[3 * k], bufs[3 * k + 1], bufs[3 * k + 2]
            if tb < _NCHUNK:
                mm = plsc.parallel_loop(0, _C // _L, unroll=8, carry=mm)(
                    minmax_body(bx, by, bz))
                if tb == _NCHUNK - 1:
                    dnums = lax.GatherDimensionNumbers(
                        offset_dims=(), collapsed_slice_dims=(0,),
                        start_index_map=(0,))

                    def allred(v, op):
                        for s in (8, 4, 2, 1):
                            perm = (ar ^ s).reshape(_L, 1)
                            shuf = lax.gather(
                                v, perm, dnums, (1,),
                                mode=lax.GatherScatterMode.PROMISE_IN_BOUNDS)
                            v = op(v, shuf)
                        return v

                    prm = []
                    for d in range(3):
                        mnv = allred(mm[d], jnp.minimum)
                        mxv = allred(mm[3 + d], jnp.maximum)
                        scv = _RES / (mxv - mnv)
                        prm.append((scv, -mnv * scv))
                    params = tuple(prm)
            else:
                plsc.parallel_loop(0, _C // _L, unroll=4,
                                   carry=jnp.int32(0))(
                    bin_body(bx, by, bz, params))
                if tb == 2 * _NCHUNK - 1:
                    def red_body(j, carry):
                        ib = j * (_L * _L) + ar * _L
                        acc = plsc.load_gather(hist, [ib])
                        for l in range(1, _L):
                            acc = acc + plsc.load_gather(hist, [ib + l])
                        featbuf[pl.ds(j * _L, _L)] = acc
                        return carry
                    plsc.parallel_loop(0, _NBINS // _L, unroll=2,
                                       carry=jnp.int32(0))(red_body)
                    bi = wid * bpw + t // (2 * _NCHUNK)
                    pltpu.sync_copy(featbuf, out_hbm.at[bi])

    return hist_kernel(xt)


def _tc_classify(feats, w, b2):
    def mm(f_ref, w_ref, b_ref, o_ref):
        acc = lax.dot_general(f_ref[...], w_ref[...],
                              (((1,), (1,)), ((), ())),
                              preferred_element_type=jnp.float32)
        o_ref[...] = acc * (1.0 / _N) + b_ref[...]

    return pl.pallas_call(
        mm,
        out_shape=jax.ShapeDtypeStruct((_B, _CLASSES), jnp.float32),
    )(feats, w, b2)


def kernel(x, W, b):
    xt = jnp.transpose(x, (2, 0, 1))
    feats = _sc_histogram(xt)
    return _tc_classify(feats, W, b.reshape(1, _CLASSES))
